# Initial kernel scaffold; baseline (speedup 1.0000x reference)
import jax
import jax.numpy as jnp
from jax.experimental import pallas as pl


def kernel(points1, points2, features1, features2, params):
    raise NotImplementedError("write your pallas kernel here")


# speedup vs baseline: 12.9311x; 12.9311x over previous
import functools

import jax
import jax.numpy as jnp
from jax.experimental import pallas as pl

_BIG = 1e10


def _iota2(shape, dim):
    return jax.lax.broadcasted_iota(jnp.int32, shape, dim)


def _fps(xyz, npoint):
    Bc, N, _ = xyz.shape
    C = 128 if N >= 128 else N
    R = N // C
    planes = xyz.transpose(0, 2, 1).reshape(Bc, 3, R, C)

    def kern(p_ref, out_ref):
        X = p_ref[:, 0, :, :]
        Y = p_ref[:, 1, :, :]
        Z = p_ref[:, 2, :, :]
        flat = (_iota2((Bc, R, C), 1) * C + _iota2((Bc, R, C), 2))
        lane = _iota2((Bc, 1, npoint), 2)

        def red(x, op):
            return op(op(x, axis=2, keepdims=True), axis=1, keepdims=True)

        def step(t, carry):
            dists, far, CX, CY, CZ = carry
            sel = flat == far
            cx = red(jnp.where(sel, X, 0.0), jnp.sum)
            cy = red(jnp.where(sel, Y, 0.0), jnp.sum)
            cz = red(jnp.where(sel, Z, 0.0), jnp.sum)
            CX = jnp.where(lane == t, cx, CX)
            CY = jnp.where(lane == t, cy, CY)
            CZ = jnp.where(lane == t, cz, CZ)
            dx = X - cx
            dy = Y - cy
            dz = Z - cz
            d = dx * dx + dy * dy + dz * dz
            dists = jnp.minimum(dists, d)
            m = red(dists, jnp.max)
            far = red(jnp.where(dists == m, flat, N), jnp.min)
            return dists, far, CX, CY, CZ

        init = (
            jnp.full((Bc, R, C), _BIG, jnp.float32),
            jnp.zeros((Bc, 1, 1), jnp.int32),
            jnp.zeros((Bc, 1, npoint), jnp.float32),
            jnp.zeros((Bc, 1, npoint), jnp.float32),
            jnp.zeros((Bc, 1, npoint), jnp.float32),
        )
        _, _, CX, CY, CZ = jax.lax.fori_loop(0, npoint, step, init)
        out_ref[:, 0:1, :] = CX
        out_ref[:, 1:2, :] = CY
        out_ref[:, 2:3, :] = CZ

    return pl.pallas_call(
        kern,
        grid=(1,),
        in_specs=[pl.BlockSpec((Bc, 3, R, C), lambda i: (0, 0, 0, 0))],
        out_specs=pl.BlockSpec((Bc, 3, npoint), lambda i: (0, 0, 0)),
        out_shape=jax.ShapeDtypeStruct((Bc, 3, npoint), jnp.float32),
    )(planes)


def _group(q_xyz, cand_xyz_t, table, ws, ns, r2, qb, self_feat=None):
    Bc, nq, _ = q_xyz.shape
    nc, Ft = table.shape[1], table.shape[2]
    nblk = nq // qb
    S = 0 if self_feat is None else self_feat.shape[2]
    gw = 3 + S + (Ft - 3)
    outF = ws[-1][0].shape[1] if ws else gw

    wargs = []
    wspecs = []
    for (W, b) in ws:
        wargs += [W, b.reshape(1, -1)]
        wspecs += [
            pl.BlockSpec(W.shape, lambda i, j: (0, 0)),
            pl.BlockSpec((1, b.shape[0]), lambda i, j: (0, 0)),
        ]
    sargs = [] if self_feat is None else [self_feat]
    sspecs = [] if self_feat is None else [
        pl.BlockSpec((1, qb, S), lambda i, j: (i, j, 0))
    ]

    def kern(q_ref, cxt_ref, tab_ref, *rest):
        out_ref = rest[-1]
        rest = rest[:-1]
        self_blk = None
        if self_feat is not None:
            self_blk = rest[0][0]
            rest = rest[1:]
        qx = q_ref[0]
        dx = qx[:, 0:1] - cxt_ref[0, 0:1, :]
        dy = qx[:, 1:2] - cxt_ref[0, 1:2, :]
        dz = qx[:, 2:3] - cxt_ref[0, 2:3, :]
        d = dx * dx + dy * dy + dz * dz
        lane = _iota2((qb, nc), 1)

        idxs = []
        if r2 is not None:
            mi = jnp.where(d <= r2, lane, nc)
            first = None
            for k in range(ns):
                cur = jnp.min(mi, axis=1, keepdims=True)
                mi = jnp.where(mi == cur, nc, mi)
                if k == 0:
                    first = jnp.where(cur == nc, 0, cur)
                    idxs.append(first)
                else:
                    idxs.append(jnp.where(cur == nc, first, cur))
        else:
            for k in range(ns):
                mval = jnp.min(d, axis=1, keepdims=True)
                cur = jnp.min(jnp.where(d == mval, lane, nc), axis=1,
                              keepdims=True)
                d = jnp.where(lane == cur, _BIG, d)
                idxs.append(cur)

        tab = tab_ref[0]
        if nc <= 512:
            idxcat = jnp.concatenate(idxs, axis=0)
            oh = (idxcat == _iota2((ns * qb, nc), 1)).astype(jnp.float32)
            rows = jnp.dot(oh, tab, preferred_element_type=jnp.float32)
        else:
            rows = jnp.concatenate(
                [jnp.dot((ix == lane).astype(jnp.float32), tab,
                         preferred_element_type=jnp.float32) for ix in idxs],
                axis=0)

        qxt = jnp.concatenate([qx] * ns, axis=0)
        parts = [rows[:, 0:3] - qxt]
        if self_blk is not None:
            parts.append(jnp.concatenate([self_blk] * ns, axis=0))
        parts.append(rows[:, 3:])
        h = jnp.concatenate(parts, axis=1)
        for (W, b) in zip(rest[0::2], rest[1::2]):
            h = jnp.maximum(
                jnp.dot(h, W[...], preferred_element_type=jnp.float32)
                + b[...], 0.0)
        h = h.reshape(ns, qb, outF)
        out_ref[0] = jnp.max(h, axis=0)

    return pl.pallas_call(
        kern,
        grid=(Bc, nblk),
        in_specs=[
            pl.BlockSpec((1, qb, 3), lambda i, j: (i, j, 0)),
            pl.BlockSpec((1, 3, nc), lambda i, j: (i, 0, 0)),
            pl.BlockSpec((1, nc, Ft), lambda i, j: (i, 0, 0)),
        ] + sspecs + wspecs,
        out_specs=pl.BlockSpec((1, qb, outF), lambda i, j: (i, j, 0)),
        out_shape=jax.ShapeDtypeStruct((Bc, nq, outF), jnp.float32),
    )(q_xyz, cand_xyz_t, table, *sargs, *wargs)


def _mlp(x, ws, relu_last=True):
    Bc, rows, _ = x.shape
    outF = ws[-1][0].shape[1]
    wargs = []
    wspecs = []
    for (W, b) in ws:
        wargs += [W, b.reshape(1, -1)]
        wspecs += [
            pl.BlockSpec(W.shape, lambda i: (0, 0)),
            pl.BlockSpec((1, b.shape[0]), lambda i: (0, 0)),
        ]

    def kern(x_ref, *rest):
        out_ref = rest[-1]
        h = x_ref[0]
        wl = list(zip(rest[0:-1:2], rest[1:-1:2]))
        for li, (W, b) in enumerate(wl):
            h = jnp.dot(h, W[...], preferred_element_type=jnp.float32) + b[...]
            if relu_last or li < len(wl) - 1:
                h = jnp.maximum(h, 0.0)
        out_ref[0] = h

    return pl.pallas_call(
        kern,
        grid=(Bc,),
        in_specs=[pl.BlockSpec((1, rows, x.shape[2]), lambda i: (i, 0, 0))]
        + wspecs,
        out_specs=pl.BlockSpec((1, rows, outF), lambda i: (i, 0, 0)),
        out_shape=jax.ShapeDtypeStruct((Bc, rows, outF), jnp.float32),
    )(x, *wargs)


def _fp_cls(fine_xyz, coarse_xyz_t, coarse_feat, fine_feat, fp_ws, cls_ws):
    Bc, N, _ = fine_xyz.shape
    M, F = coarse_feat.shape[1], coarse_feat.shape[2]
    S = fine_feat.shape[2]
    qb = 512
    nblk = N // qb
    layers = list(fp_ws) + list(cls_ws)
    n_relu = len(layers) - 1
    wargs = []
    wspecs = []
    for (W, b) in layers:
        wargs += [W, b.reshape(1, -1)]
        wspecs += [
            pl.BlockSpec(W.shape, lambda i, j: (0, 0)),
            pl.BlockSpec((1, b.shape[0]), lambda i, j: (0, 0)),
        ]

    def kern(fx_ref, cxt_ref, cf_ref, ff_ref, *rest):
        out_ref = rest[-1]
        wl = list(zip(rest[0:-1:2], rest[1:-1:2]))
        qx = fx_ref[0]
        dx = qx[:, 0:1] - cxt_ref[0, 0:1, :]
        dy = qx[:, 1:2] - cxt_ref[0, 1:2, :]
        dz = qx[:, 2:3] - cxt_ref[0, 2:3, :]
        d = dx * dx + dy * dy + dz * dz
        lane = _iota2((qb, M), 1)
        Wm = jnp.zeros((qb, M), jnp.float32)
        wsum = jnp.zeros((qb, 1), jnp.float32)
        for k in range(3):
            mval = jnp.min(d, axis=1, keepdims=True)
            cur = jnp.min(jnp.where(d == mval, lane, M), axis=1,
                          keepdims=True)
            d = jnp.where(lane == cur, _BIG, d)
            wk = 1.0 / (mval + 1e-10)
            Wm = Wm + wk * (lane == cur).astype(jnp.float32)
            wsum = wsum + wk
        Wm = Wm / wsum
        interp = jnp.dot(Wm, cf_ref[0], preferred_element_type=jnp.float32)
        h = jnp.concatenate([interp, ff_ref[0]], axis=1)
        for li, (W, b) in enumerate(wl):
            h = jnp.dot(h, W[...], preferred_element_type=jnp.float32) + b[...]
            if li < n_relu:
                h = jnp.maximum(h, 0.0)
        out_ref[0] = h

    return pl.pallas_call(
        kern,
        grid=(Bc, nblk),
        in_specs=[
            pl.BlockSpec((1, qb, 3), lambda i, j: (i, j, 0)),
            pl.BlockSpec((1, 3, M), lambda i, j: (i, 0, 0)),
            pl.BlockSpec((1, M, F), lambda i, j: (i, 0, 0)),
            pl.BlockSpec((1, qb, S), lambda i, j: (i, j, 0)),
        ] + wspecs,
        out_specs=pl.BlockSpec((1, qb, 3), lambda i, j: (i, j, 0)),
        out_shape=jax.ShapeDtypeStruct((Bc, N, 3), jnp.float32),
    )(fine_xyz, coarse_xyz_t, coarse_feat, fine_feat, *wargs)


def kernel(points1, points2, features1, features2, params):
    P = params
    f1t = features1.transpose(0, 2, 1)
    f2t = features2.transpose(0, 2, 1)
    pts = jnp.concatenate([points1, points2], axis=0)
    ft = jnp.concatenate([f1t, f2t], axis=0)

    nx_t = _fps(pts, 256)
    nx = nx_t.transpose(0, 2, 1)
    table1 = jnp.concatenate([pts, ft], axis=-1)
    f_sc1 = _group(nx, pts.transpose(0, 2, 1), table1, P['sc1'],
                   ns=16, r2=1.0, qb=128)
    B = points1.shape[0]
    p12, p22 = nx[:B], nx[B:]
    p12t, p22t = nx_t[:B], nx_t[B:]
    f12, f22 = f_sc1[:B], f_sc1[B:]

    table_fe = jnp.concatenate([p22, f22], axis=-1)
    emb = _group(p12, p22t, table_fe, P['fe'],
                 ns=64, r2=None, qb=64, self_feat=f12)

    p13t = _fps(p12, 64)
    p13 = p13t.transpose(0, 2, 1)
    table2 = jnp.concatenate([p12, emb], axis=-1)
    f13 = _group(p13, p12t, table2, P['sc2'], ns=8, r2=4.0, qb=64)

    p14t = _fps(p13, 16)
    p14 = p14t.transpose(0, 2, 1)
    table3 = jnp.concatenate([p13, f13], axis=-1)
    f14 = _group(p14, p13t, table3, P['sc3'], ns=8, r2=16.0, qb=16)

    t_up1 = jnp.concatenate([p14, f14], axis=-1)
    g1 = _group(p13, p14t, t_up1, [], ns=8, r2=None, qb=64)
    nf13 = _mlp(jnp.concatenate([g1, f13], axis=-1), P['up1_mlp2'])

    t_up2 = jnp.concatenate([p13, nf13], axis=-1)
    g2 = _group(p12, p13t, t_up2, P['up2_mlp1'], ns=8, r2=None, qb=256)
    skip_t = jnp.concatenate([f12, emb], axis=-1)
    nf12 = _mlp(jnp.concatenate([g2, skip_t], axis=-1), P['up2_mlp2'])

    out = _fp_cls(points1, p12t, nf12, f1t, P['fp'], P['cls'])
    return out.transpose(0, 2, 1)
